# Initial kernel scaffold; baseline (speedup 1.0000x reference)
import jax
import jax.numpy as jnp
from jax.experimental import pallas as pl


def kernel(x, edge_index, train_edge_id, W1, b1, fcW, fcb):
    raise NotImplementedError("write your pallas kernel here")


# speedup vs baseline: 11.7435x; 11.7435x over previous
import functools

import jax
import jax.numpy as jnp
from jax import lax
from jax.experimental import pallas as pl
from jax.experimental.pallas import tpu as pltpu
from jax.experimental.pallas import tpu_sc as plsc

NCORES = 2
NSUB = 16
NW = NCORES * NSUB
LANES = 16
CH = 128


def _mesh():
    return plsc.VectorSubcoreMesh(core_axis_name="c", subcore_axis_name="s",
                                  num_cores=NCORES, num_subcores=NSUB)


def _make_sc_deg(n_pad, n_edges):
    ept = n_edges // NW
    B = 128
    nfull = ept // B
    rem = ept - nfull * B
    rpt = n_pad // NSUB
    nch = rpt // CH

    @functools.partial(
        pl.kernel,
        out_type=jax.ShapeDtypeStruct((NCORES * n_pad, LANES), jnp.float32),
        mesh=_mesh(),
        scratch_types=[
            pltpu.VMEM((B,), jnp.int32),
            pltpu.VMEM((rem,), jnp.int32),
            pltpu.VMEM((B, LANES), jnp.float32),
            pltpu.VMEM((CH, LANES), jnp.float32),
            pltpu.VMEM_SHARED((n_pad, LANES), jnp.float32),
        ],
    )
    def deg_kernel(dst_hbm, out_hbm, idx_v, idxr_v, ones_v, stage_v, acc_sh):
        cid = lax.axis_index("c")
        sid = lax.axis_index("s")
        wid = sid * NCORES + cid

        ones16 = jnp.ones((LANES,), jnp.float32)
        zero16 = jnp.zeros((LANES,), jnp.float32)

        def fill_ones(i, _):
            ones_v[i, :] = ones16
            return 0
        lax.fori_loop(0, B, fill_ones, 0)

        def fill_zero(i, _):
            stage_v[i, :] = zero16
            return 0
        lax.fori_loop(0, CH, fill_zero, 0)

        def zchunk(i, _):
            pltpu.sync_copy(stage_v, acc_sh.at[pl.ds(sid * rpt + i * CH, CH)])
            return 0
        lax.fori_loop(0, nch, zchunk, 0)
        plsc.subcore_barrier()

        base = wid * ept

        def batch(i, _):
            pltpu.sync_copy(dst_hbm.at[pl.ds(base + i * B, B)], idx_v)
            pltpu.sync_copy(ones_v, acc_sh.at[idx_v], add=True)
            return 0
        lax.fori_loop(0, nfull, batch, 0)
        if rem:
            pltpu.sync_copy(dst_hbm.at[pl.ds(base + nfull * B, rem)], idxr_v)
            pltpu.sync_copy(ones_v.at[pl.ds(0, rem)], acc_sh.at[idxr_v], add=True)

        plsc.subcore_barrier()

        def ochunk(i, _):
            r = sid * rpt + i * CH
            pltpu.sync_copy(acc_sh.at[pl.ds(r, CH)], stage_v)
            pltpu.sync_copy(stage_v, out_hbm.at[pl.ds(cid * n_pad + r, CH)])
            return 0
        lax.fori_loop(0, nch, ochunk, 0)

    return deg_kernel


def _make_sc_scatter(n_pad, n_edges, d):
    ept = n_edges // NW
    B = 128
    nfull = ept // B
    rem = ept - nfull * B
    rpt = n_pad // NSUB
    nch = rpt // CH

    @functools.partial(
        pl.kernel,
        out_type=jax.ShapeDtypeStruct((NCORES * n_pad, d), jnp.float32),
        mesh=_mesh(),
        scratch_types=[
            pltpu.VMEM((B,), jnp.int32),
            pltpu.VMEM((B,), jnp.int32),
            pltpu.VMEM((rem,), jnp.int32),
            pltpu.VMEM((rem,), jnp.int32),
            pltpu.VMEM((B, d), jnp.float32),
            pltpu.VMEM((rem, d), jnp.float32),
            pltpu.VMEM_SHARED((n_pad, d), jnp.float32),
            pltpu.SemaphoreType.DMA,
        ],
    )
    def scat_kernel(src_hbm, dst_hbm, y_hbm, out_hbm,
                    sidx_v, didx_v, sidxr_v, didxr_v, rows_v, rowsr_v,
                    acc_sh, sem):
        cid = lax.axis_index("c")
        sid = lax.axis_index("s")
        wid = sid * NCORES + cid

        zero16 = jnp.zeros((LANES,), jnp.float32)
        nvec = d // LANES

        def fill_zero(i, _):
            def inner(j, _):
                rows_v[i, pl.ds(j * LANES, LANES)] = zero16
                return 0
            lax.fori_loop(0, nvec, inner, 0)
            return 0
        lax.fori_loop(0, CH, fill_zero, 0)

        def zchunk(i, _):
            pltpu.sync_copy(rows_v, acc_sh.at[pl.ds(sid * rpt + i * CH, CH)])
            return 0
        lax.fori_loop(0, nch, zchunk, 0)
        plsc.subcore_barrier()

        base = wid * ept

        def batch(i, _):
            off = base + i * B
            pltpu.sync_copy(src_hbm.at[pl.ds(off, B)], sidx_v)
            pltpu.sync_copy(dst_hbm.at[pl.ds(off, B)], didx_v)
            pltpu.async_copy(y_hbm.at[sidx_v], rows_v, sem).wait()
            pltpu.sync_copy(rows_v, acc_sh.at[didx_v], add=True)
            return 0
        lax.fori_loop(0, nfull, batch, 0)
        if rem:
            off = base + nfull * B
            pltpu.sync_copy(src_hbm.at[pl.ds(off, rem)], sidxr_v)
            pltpu.sync_copy(dst_hbm.at[pl.ds(off, rem)], didxr_v)
            pltpu.async_copy(y_hbm.at[sidxr_v], rowsr_v, sem).wait()
            pltpu.sync_copy(rowsr_v, acc_sh.at[didxr_v], add=True)

        plsc.subcore_barrier()

        def ochunk(i, _):
            r = sid * rpt + i * CH
            pltpu.sync_copy(acc_sh.at[pl.ds(r, CH)], rows_v)
            pltpu.sync_copy(rows_v, out_hbm.at[pl.ds(cid * n_pad + r, CH)])
            return 0
        lax.fori_loop(0, nch, ochunk, 0)

    return scat_kernel


def _make_sc_final(n_edges, n_train, d):
    tpt = n_train // NW
    B = 128
    nb = tpt // B
    nj = B // LANES

    @functools.partial(
        pl.kernel,
        out_type=jax.ShapeDtypeStruct((n_train, LANES), jnp.float32),
        mesh=_mesh(),
        scratch_types=[
            pltpu.VMEM((B,), jnp.int32),
            pltpu.VMEM((B,), jnp.int32),
            pltpu.VMEM((B,), jnp.int32),
            pltpu.VMEM((B, 128), jnp.float32),
            pltpu.VMEM((B, 128), jnp.float32),
            pltpu.VMEM((B, LANES), jnp.float32),
            pltpu.SemaphoreType.DMA,
            pltpu.SemaphoreType.DMA,
        ],
    )
    def fin_kernel(e0_hbm, e1_hbm, tid_hbm, a_hbm, b_hbm, out_hbm,
                   tid_v, s_v, d_v, ar_v, br_v, ob_v, sem, sem2):
        cid = lax.axis_index("c")
        sid = lax.axis_index("s")
        wid = sid * NCORES + cid
        base = wid * tpt

        def batch(i, _):
            off = base + i * B
            pltpu.sync_copy(tid_hbm.at[pl.ds(off, B)], tid_v)

            def pick(j, _):
                t16 = tid_v[pl.ds(j * LANES, LANES)]
                pltpu.async_copy(e0_hbm.at[t16],
                                 s_v.at[pl.ds(j * LANES, LANES)], sem).wait()
                pltpu.async_copy(e1_hbm.at[t16],
                                 d_v.at[pl.ds(j * LANES, LANES)], sem).wait()
                return 0
            lax.fori_loop(0, nj, pick, 0)

            pltpu.async_copy(a_hbm.at[s_v], ar_v, sem).wait()
            pltpu.async_copy(b_hbm.at[d_v], br_v, sem).wait()

            def add_row(k, _):
                ob_v[k, :] = ar_v[k, pl.ds(0, LANES)] + br_v[k, pl.ds(0, LANES)]
                return 0
            lax.fori_loop(0, B, add_row, 0)
            pltpu.sync_copy(ob_v, out_hbm.at[pl.ds(off, B)])
            return 0
        lax.fori_loop(0, nb, batch, 0)

    return fin_kernel


def _tc_y(degacc, x2, w1t, n_nodes, d, blk=1024):
    grid = (n_nodes // blk,)

    def body(da_ref, x_ref, w_ref, y_ref):
        deg = (da_ref[0, :, 0:1] + da_ref[1, :, 0:1]) * 0.5 + 1.0
        dinv = lax.rsqrt(deg)
        xw = jnp.dot(x_ref[...], w_ref[...],
                     preferred_element_type=jnp.float32)
        y_ref[...] = dinv * xw

    return pl.pallas_call(
        body,
        grid=grid,
        in_specs=[
            pl.BlockSpec((2, blk, LANES), lambda i: (0, i, 0)),
            pl.BlockSpec((blk, d), lambda i: (i, 0)),
            pl.BlockSpec((d, d), lambda i: (0, 0)),
        ],
        out_specs=pl.BlockSpec((blk, d), lambda i: (i, 0)),
        out_shape=jax.ShapeDtypeStruct((n_nodes, d), jnp.float32),
    )(degacc, x2, w1t)


def _tc_combine(degacc, sacc, y, b1r, wpa, wpb, fcba, n_nodes, d, blk=1024):
    grid = (n_nodes // blk,)

    def body(da_ref, s_ref, y_ref, b_ref, wa_ref, wb_ref, fb_ref,
             a_ref, bt_ref):
        deg = (da_ref[0, :, 0:1] + da_ref[1, :, 0:1]) * 0.5 + 1.0
        dinv = lax.rsqrt(deg)
        s = s_ref[0] + s_ref[1] + y_ref[...]
        h = jnp.maximum(dinv * s + b_ref[...], 0.0)
        a_ref[...] = jnp.dot(h, wa_ref[...],
                             preferred_element_type=jnp.float32) + fb_ref[...]
        bt_ref[...] = jnp.dot(h, wb_ref[...],
                              preferred_element_type=jnp.float32)

    return pl.pallas_call(
        body,
        grid=grid,
        in_specs=[
            pl.BlockSpec((2, blk, LANES), lambda i: (0, i, 0)),
            pl.BlockSpec((2, blk, d), lambda i: (0, i, 0)),
            pl.BlockSpec((blk, d), lambda i: (i, 0)),
            pl.BlockSpec((1, d), lambda i: (0, 0)),
            pl.BlockSpec((d, d), lambda i: (0, 0)),
            pl.BlockSpec((d, d), lambda i: (0, 0)),
            pl.BlockSpec((1, d), lambda i: (0, 0)),
        ],
        out_specs=[
            pl.BlockSpec((blk, d), lambda i: (i, 0)),
            pl.BlockSpec((blk, d), lambda i: (i, 0)),
        ],
        out_shape=[
            jax.ShapeDtypeStruct((n_nodes, d), jnp.float32),
            jax.ShapeDtypeStruct((n_nodes, d), jnp.float32),
        ],
    )(degacc, sacc, y, b1r, wpa, wpb, fcba)


def kernel(x, edge_index, train_edge_id, W1, b1, fcW, fcb):
    n, _, d = x.shape
    e = edge_index.shape[1]
    nt = train_edge_id.shape[0]
    nc = fcW.shape[0]
    n_pad = ((n + NSUB * CH - 1) // (NSUB * CH)) * (NSUB * CH)

    x2 = jnp.pad(x.reshape(n, d), ((0, n_pad - n), (0, 0)))
    e0 = edge_index[0]
    e1 = edge_index[1]

    wpa = jnp.zeros((d, d), jnp.float32).at[:, :nc].set(fcW[:, :d].T)
    wpb = jnp.zeros((d, d), jnp.float32).at[:, :nc].set(fcW[:, d:].T)
    fcba = jnp.zeros((1, d), jnp.float32).at[0, :nc].set(fcb)
    b1r = b1.reshape(1, d)
    w1t = W1.T

    degacc2 = _make_sc_deg(n_pad, e)(e1) * 2.0
    degacc2 = degacc2.reshape(2, n_pad, LANES)
    y = _tc_y(degacc2, x2, w1t, n_pad, d)
    y2 = y * 2.0
    sacc = _make_sc_scatter(n_pad, e, d)(e0, e1, y2) * 0.5
    sacc = sacc.reshape(2, n_pad, d)
    a_tab, b_tab = _tc_combine(degacc2, sacc, y, b1r, wpa, wpb, fcba, n_pad, d)
    out16 = _make_sc_final(e, nt, d)(e0, e1, train_edge_id,
                                     a_tab * 2.0, b_tab * 2.0)
    return out16[:, :nc] * 0.5
